# baseline (speedup 1.0000x reference)
import jax
import jax.numpy as jnp
from jax.experimental import pallas as pl
import jax.experimental.pallas.tpu as pltpu

BATCH = 2048
DIM = 2048
NUM_EXPERTS = 8
ATTN_GAIN_MEAN = 1.0 + 32.0 / 2048.0
KT = 256
JT = 512


def _fused_kernel(x_ref, gw_ref, gb_ref, ex_ref, out_ref,
                  xbf_ref, w_ref):
    j = pl.program_id(0)
    k = pl.program_id(1)

    @pl.when(jnp.logical_and(j == 0, k == 0))
    def _init():
        x = x_ref[...]
        mu = jnp.mean(x, axis=1, keepdims=True)
        xc = x - mu
        std = jnp.sqrt(jnp.mean(xc * xc, axis=1, keepdims=True)) + 1e-6
        xbf_ref[...] = (xc / std).astype(jnp.bfloat16)
        logits = gw_ref[0:1, :] + ATTN_GAIN_MEAN * gw_ref[1:2, :] + gb_ref[...]
        m = jnp.max(logits, axis=1, keepdims=True)
        e = jnp.exp(logits - m)
        w_ref[...] = e / jnp.sum(e, axis=1, keepdims=True)

    ex = ex_ref[...]
    comb = w_ref[0, 0] * ex[0]
    for e_idx in range(1, NUM_EXPERTS):
        comb = comb + w_ref[0, e_idx] * ex[e_idx]

    xk = xbf_ref[:, pl.ds(k * KT, KT)]
    acc = jnp.dot(xk, comb.astype(jnp.bfloat16),
                  preferred_element_type=jnp.float32)

    @pl.when(k == 0)
    def _first():
        out_ref[...] = acc

    @pl.when(k > 0)
    def _rest():
        out_ref[...] += acc


@jax.jit
def kernel(query_embedding, gate_W, gate_b, experts):
    nj = DIM // JT
    nk = DIM // KT
    gate_b2 = gate_b.reshape(1, NUM_EXPERTS)
    return pl.pallas_call(
        _fused_kernel,
        grid=(nj, nk),
        in_specs=[
            pl.BlockSpec((BATCH, DIM), lambda j, k: (0, 0)),
            pl.BlockSpec((12, NUM_EXPERTS), lambda j, k: (0, 0)),
            pl.BlockSpec((1, NUM_EXPERTS), lambda j, k: (0, 0)),
            pl.BlockSpec((NUM_EXPERTS, KT, JT), lambda j, k: (0, k, j)),
        ],
        out_specs=pl.BlockSpec((BATCH, JT), lambda j, k: (0, j)),
        out_shape=jax.ShapeDtypeStruct((BATCH, DIM), jnp.float32),
        scratch_shapes=[pltpu.VMEM((BATCH, DIM), jnp.bfloat16),
                        pltpu.VMEM((1, NUM_EXPERTS), jnp.float32)],
    )(query_embedding, gate_W, gate_b2, experts)


# speedup vs baseline: 1.1270x; 1.1270x over previous
import jax
import jax.numpy as jnp
from jax.experimental import pallas as pl
import jax.experimental.pallas.tpu as pltpu

BATCH = 2048
DIM = 2048
NUM_EXPERTS = 8
ATTN_GAIN_MEAN = 1.0 + 32.0 / 2048.0
KT = 256
JT = 1024
NJ = DIM // JT
NK = DIM // KT
NSTEPS = NJ * NK


def _ex_copy(ex_hbm, ebuf_ref, sem, slot, k, j):
    return pltpu.make_async_copy(
        ex_hbm.at[:, pl.ds(k * KT, KT), pl.ds(j * JT, JT)],
        ebuf_ref.at[slot],
        sem.at[slot],
    )


def _fused_kernel(x_ref, gw_ref, gb_ref, ex_hbm, out_ref,
                  ebuf_ref, w_ref, sem):
    s = pl.program_id(0)
    k = s // NJ
    j = jax.lax.rem(s, NJ)
    slot = jax.lax.rem(s, 2)

    @pl.when(s == 0)
    def _init():
        _ex_copy(ex_hbm, ebuf_ref, sem, 0, 0, 0).start()
        chunk = 256
        for c in range(BATCH // chunk):
            x = x_ref[pl.ds(c * chunk, chunk), :]
            mu = jnp.mean(x, axis=1, keepdims=True)
            xc = x - mu
            std = jnp.sqrt(jnp.mean(xc * xc, axis=1, keepdims=True)) + 1e-6
            x_ref[pl.ds(c * chunk, chunk), :] = xc / std
        logits = gw_ref[0:1, :] + ATTN_GAIN_MEAN * gw_ref[1:2, :] + gb_ref[...]
        m = jnp.max(logits, axis=1, keepdims=True)
        e = jnp.exp(logits - m)
        w_ref[...] = e / jnp.sum(e, axis=1, keepdims=True)

    @pl.when(s + 1 < NSTEPS)
    def _prefetch():
        sn = s + 1
        _ex_copy(ex_hbm, ebuf_ref, sem, 1 - slot, sn // NJ,
                 jax.lax.rem(sn, NJ)).start()

    _ex_copy(ex_hbm, ebuf_ref, sem, slot, k, j).wait()

    ex = ebuf_ref[slot]
    comb = w_ref[0, 0] * ex[0]
    for e_idx in range(1, NUM_EXPERTS):
        comb = comb + w_ref[0, e_idx] * ex[e_idx]

    comb_bf = comb.astype(jnp.bfloat16)
    half = BATCH // 2
    for r in range(2):
        xk = x_ref[pl.ds(r * half, half), pl.ds(k * KT, KT)]
        acc = jnp.dot(xk.astype(jnp.bfloat16), comb_bf,
                      preferred_element_type=jnp.float32)

        @pl.when(k == 0)
        def _first():
            out_ref[pl.ds(r * half, half), pl.ds(j * JT, JT)] = acc

        @pl.when(k > 0)
        def _rest():
            out_ref[pl.ds(r * half, half), pl.ds(j * JT, JT)] += acc


@jax.jit
def kernel(query_embedding, gate_W, gate_b, experts):
    gate_b2 = gate_b.reshape(1, NUM_EXPERTS)
    return pl.pallas_call(
        _fused_kernel,
        grid=(NSTEPS,),
        in_specs=[
            pl.BlockSpec((BATCH, DIM), lambda s: (0, 0)),
            pl.BlockSpec((12, NUM_EXPERTS), lambda s: (0, 0)),
            pl.BlockSpec((1, NUM_EXPERTS), lambda s: (0, 0)),
            pl.BlockSpec(memory_space=pl.ANY),
        ],
        out_specs=pl.BlockSpec((BATCH, DIM), lambda s: (0, 0)),
        out_shape=jax.ShapeDtypeStruct((BATCH, DIM), jnp.float32),
        scratch_shapes=[
            pltpu.VMEM((2, NUM_EXPERTS, KT, JT), jnp.float32),
            pltpu.VMEM((1, NUM_EXPERTS), jnp.float32),
            pltpu.SemaphoreType.DMA((2,)),
        ],
    )(query_embedding, gate_W, gate_b2, experts)


# speedup vs baseline: 1.1635x; 1.0324x over previous
import jax
import jax.numpy as jnp
from jax.experimental import pallas as pl
import jax.experimental.pallas.tpu as pltpu

BATCH = 2048
DIM = 2048
NUM_EXPERTS = 8
ATTN_GAIN_MEAN = 1.0 + 32.0 / 2048.0
KT = 1024
JT = 512
NJ = DIM // JT
NK = DIM // KT
NSTEPS = NJ * NK
PREP_CHUNK = 256


def _prep_kernel(x_ref, gw_ref, gb_ref, xbf_ref, w_ref):
    c = pl.program_id(0)

    @pl.when(c == 0)
    def _gate():
        logits = gw_ref[0:1, :] + ATTN_GAIN_MEAN * gw_ref[1:2, :] + gb_ref[...]
        m = jnp.max(logits, axis=1, keepdims=True)
        e = jnp.exp(logits - m)
        w_ref[...] = e / jnp.sum(e, axis=1, keepdims=True)

    x = x_ref[...]
    mu = jnp.mean(x, axis=1, keepdims=True)
    xc = x - mu
    std = jnp.sqrt(jnp.mean(xc * xc, axis=1, keepdims=True)) + 1e-6
    xbf_ref[...] = (xc / std).astype(jnp.bfloat16)


def _ex_copy(ex_hbm, ebuf_ref, sem, slot, k, j):
    return pltpu.make_async_copy(
        ex_hbm.at[:, pl.ds(k * KT, KT), pl.ds(j * JT, JT)],
        ebuf_ref.at[slot],
        sem.at[slot],
    )


def _main_kernel(xbf_ref, w_ref, ex_hbm, out_ref, ebuf_ref, sem):
    s = pl.program_id(0)
    j = s // NK
    k = jax.lax.rem(s, NK)
    slot = jax.lax.rem(s, 2)

    @pl.when(s == 0)
    def _first_copy():
        _ex_copy(ex_hbm, ebuf_ref, sem, 0, 0, 0).start()

    @pl.when(s + 1 < NSTEPS)
    def _prefetch():
        sn = s + 1
        _ex_copy(ex_hbm, ebuf_ref, sem, 1 - slot,
                 jax.lax.rem(sn, NK), sn // NK).start()

    _ex_copy(ex_hbm, ebuf_ref, sem, slot, k, j).wait()

    ex = ebuf_ref[slot]
    comb = w_ref[0, 0] * ex[0]
    for e_idx in range(1, NUM_EXPERTS):
        comb = comb + w_ref[0, e_idx] * ex[e_idx]

    xk = xbf_ref[:, pl.ds(k * KT, KT)]
    acc = jnp.dot(xk, comb.astype(jnp.bfloat16),
                  preferred_element_type=jnp.float32)

    @pl.when(k == 0)
    def _fst():
        out_ref[...] = acc

    @pl.when(k > 0)
    def _rst():
        out_ref[...] += acc


@jax.jit
def kernel(query_embedding, gate_W, gate_b, experts):
    gate_b2 = gate_b.reshape(1, NUM_EXPERTS)
    xbf, w = pl.pallas_call(
        _prep_kernel,
        grid=(BATCH // PREP_CHUNK,),
        in_specs=[
            pl.BlockSpec((PREP_CHUNK, DIM), lambda c: (c, 0)),
            pl.BlockSpec((12, NUM_EXPERTS), lambda c: (0, 0)),
            pl.BlockSpec((1, NUM_EXPERTS), lambda c: (0, 0)),
        ],
        out_specs=[
            pl.BlockSpec((PREP_CHUNK, DIM), lambda c: (c, 0)),
            pl.BlockSpec((1, NUM_EXPERTS), lambda c: (0, 0)),
        ],
        out_shape=[
            jax.ShapeDtypeStruct((BATCH, DIM), jnp.bfloat16),
            jax.ShapeDtypeStruct((1, NUM_EXPERTS), jnp.float32),
        ],
    )(query_embedding, gate_W, gate_b2)

    return pl.pallas_call(
        _main_kernel,
        grid=(NSTEPS,),
        in_specs=[
            pl.BlockSpec((BATCH, DIM), lambda s: (0, 0)),
            pl.BlockSpec((1, NUM_EXPERTS), lambda s: (0, 0)),
            pl.BlockSpec(memory_space=pl.ANY),
        ],
        out_specs=pl.BlockSpec((BATCH, JT), lambda s: (0, s // NK)),
        out_shape=jax.ShapeDtypeStruct((BATCH, DIM), jnp.float32),
        scratch_shapes=[
            pltpu.VMEM((2, NUM_EXPERTS, KT, JT), jnp.float32),
            pltpu.SemaphoreType.DMA((2,)),
        ],
    )(xbf, w, experts)
